# Initial kernel scaffold; baseline (speedup 1.0000x reference)
import jax
import jax.numpy as jnp
from jax import lax
from jax.experimental import pallas as pl
from jax.experimental.pallas import tpu as pltpu
from jax.experimental.pallas import tpu_sc as plsc

T = 204800
B = 4096
D = 32
C = 4
NC = 2
NS = 16
NW = NC * NS
W = T // NW
CH = 1280
NCHUNK = W // CH
SUB = 128
NSUB = CH // SUB
SEGROWS = W // SUB
CNTW = W + 8
BPW = B // NW


def _sc_body(text_hbm, offs_hbm, emb_hbm, part_hbm, inv_hbm,
             offs_v, pos2d, ones_v, cnt_v, seg2d, idx_v, rows_v, inv_v,
             cnt_sc, acc_sc, sem):
    cid = lax.axis_index("c")
    sid = lax.axis_index("s")
    wid = cid * NS + sid
    wbase = wid * W

    zeros16f = jnp.zeros((16,), jnp.float32)
    zeros16i = jnp.zeros((16,), jnp.int32)

    def _z1(i, carry):
        rows_v[i, pl.ds(0, 16)] = zeros16f
        rows_v[i, pl.ds(16, 16)] = zeros16f
        return carry
    lax.fori_loop(0, B // NS, _z1, 0)
    pltpu.sync_copy(rows_v.at[pl.ds(0, B // NS)],
                    acc_sc.at[pl.ds(sid * (B // NS), B // NS)])

    pltpu.sync_copy(offs_hbm, offs_v.at[pl.ds(0, B)])
    offs_v[pl.ds(B, 16)] = jnp.full((16,), T, jnp.int32)

    def _z2(i, carry):
        cnt_v[pl.ds(i * 16, 16)] = zeros16i
        return carry
    lax.fori_loop(0, W // 16, _z2, 0)
    rowoff = sid * CNTW
    pltpu.sync_copy(cnt_v, cnt_sc.at[pl.ds(rowoff, W)])

    def _pos(r, carry):
        for j in range(8):
            v = offs_v[pl.ds(r * 128 + j * 16, 16)]
            p = jnp.clip(v - wbase, 0, W) + rowoff
            pos2d[r, pl.ds(j * 16, 16)] = p
        return carry
    lax.fori_loop(0, B // 128, _pos, 0)

    for j in range(8):
        ones_v[pl.ds(j * 16, 16)] = jnp.full((16,), 1, jnp.int32)

    plsc.subcore_barrier()

    def _hist(j, carry):
        pltpu.sync_copy(ones_v, cnt_sc.at[pos2d.at[j]], add=True)
        return carry
    lax.fori_loop(0, B // SUB, _hist, 0)
    pltpu.sync_copy(cnt_sc.at[pl.ds(rowoff, W)], cnt_v)

    def _cs(r, carry):
        for j in range(8):
            v = cnt_v[pl.ds(r * 128 + j * 16, 16)]
            seg2d[r, pl.ds(j * 16, 16)] = plsc.cumsum(v) + carry - 1
            carry = carry + jnp.sum(v)
        return carry
    lax.fori_loop(0, SEGROWS, _cs, jnp.int32(0))

    bbase = wid * BPW
    for j in range(BPW // 16):
        a = offs_v[pl.ds(bbase + j * 16, 16)]
        b2 = offs_v[pl.ds(bbase + j * 16 + 1, 16)]
        cntf = (b2 - a).astype(jnp.float32)
        inv_v[pl.ds(j * 16, 16)] = 1.0 / jnp.maximum(cntf, 1.0)
    pltpu.sync_copy(inv_v, inv_hbm.at[pl.ds(bbase, BPW)])

    def _chunk(k, carry):
        tb = wbase + k * CH
        pltpu.sync_copy(text_hbm.at[pl.ds(tb, CH)], idx_v)
        descs = [
            pltpu.async_copy(emb_hbm.at[idx_v.at[pl.ds(j * SUB, SUB)]],
                             rows_v.at[pl.ds(j * SUB, SUB)], sem)
            for j in range(NSUB)
        ]
        for d in descs:
            d.wait()
        for j in range(NSUB):
            pltpu.sync_copy(rows_v.at[pl.ds(j * SUB, SUB)],
                            acc_sc.at[seg2d.at[k * NSUB + j]], add=True)
        return carry
    lax.fori_loop(0, NCHUNK, _chunk, 0)

    plsc.subcore_barrier()

    pltpu.sync_copy(acc_sc.at[pl.ds(sid * (B // NS), B // NS)],
                    rows_v.at[pl.ds(0, B // NS)])
    rowbase = cid * B + sid * (B // NS)
    pltpu.sync_copy(rows_v.at[pl.ds(0, B // NS)],
                    part_hbm.at[pl.ds(rowbase, B // NS)])


_sc_embed = pl.kernel(
    _sc_body,
    out_type=(
        jax.ShapeDtypeStruct((NC * B, D), jnp.float32),
        jax.ShapeDtypeStruct((B,), jnp.float32),
    ),
    mesh=plsc.VectorSubcoreMesh(core_axis_name="c", subcore_axis_name="s",
                                num_cores=NC, num_subcores=NS),
    scratch_types=[
        pltpu.VMEM((B + 16,), jnp.int32),
        pltpu.VMEM((B // SUB, SUB), jnp.int32),
        pltpu.VMEM((SUB,), jnp.int32),
        pltpu.VMEM((W,), jnp.int32),
        pltpu.VMEM((SEGROWS, SUB), jnp.int32),
        pltpu.VMEM((CH,), jnp.int32),
        pltpu.VMEM((CH, D), jnp.float32),
        pltpu.VMEM((BPW,), jnp.float32),
        pltpu.VMEM_SHARED((NS * CNTW,), jnp.int32),
        pltpu.VMEM_SHARED((B, D), jnp.float32),
        pltpu.SemaphoreType.DMA,
    ],
)


def _tc_body(p_ref, inv_ref, fcw_ref, bias_ref, out_ref):
    sums = p_ref[pl.ds(0, B), :] + p_ref[pl.ds(B, B), :]
    means = sums * inv_ref[...]
    out_ref[...] = lax.dot_general(
        means, fcw_ref[...], (((1,), (1,)), ((), ())),
        preferred_element_type=jnp.float32) + bias_ref[...]


_tc_head = pl.pallas_call(
    _tc_body,
    out_shape=jax.ShapeDtypeStruct((B, C), jnp.float32),
)


def kernel(text, offsets, emb_weight, fc_weight, fc_bias):
    part, inv = _sc_embed(text, offsets, emb_weight)
    return _tc_head(part, inv.reshape(B, 1), fc_weight, fc_bias.reshape(1, C))


# speedup vs baseline: 37.6800x; 37.6800x over previous
import jax
import jax.numpy as jnp
from jax import lax
from jax.experimental import pallas as pl
from jax.experimental.pallas import tpu as pltpu
from jax.experimental.pallas import tpu_sc as plsc

T = 204800
B = 4096
D = 32
C = 4
NC = 2
NS = 16
NW = NC * NS
W = T // NW
CH = 1280
NCHUNK = W // CH
SUB = 128
NSUB = CH // SUB
SEGROWS = W // SUB
CNTW = W + 8
BPW = B // NW


def _sc_body(text_hbm, offs_hbm, emb_hbm, part_hbm, inv_hbm,
             offs_v, pos2d, ones_v, cnt_v, seg2d, idx_v, rows_v, inv_v,
             cnt_sc, acc_sc, sem):
    cid = lax.axis_index("c")
    sid = lax.axis_index("s")
    wid = cid * NS + sid
    wbase = wid * W

    zeros16f = jnp.zeros((16,), jnp.float32)
    zeros16i = jnp.zeros((16,), jnp.int32)

    def _z1(i, carry):
        rows_v[i, pl.ds(0, 16)] = zeros16f
        rows_v[i, pl.ds(16, 16)] = zeros16f
        return carry
    lax.fori_loop(0, B // NS, _z1, 0)
    pltpu.sync_copy(rows_v.at[pl.ds(0, B // NS)],
                    acc_sc.at[pl.ds(sid * (B // NS), B // NS)])

    pltpu.sync_copy(offs_hbm, offs_v.at[pl.ds(0, B)])
    offs_v[pl.ds(B, 16)] = jnp.full((16,), T, jnp.int32)

    def _z2(i, carry):
        cnt_v[pl.ds(i * 16, 16)] = zeros16i
        return carry
    lax.fori_loop(0, W // 16, _z2, 0)
    rowoff = sid * CNTW
    pltpu.sync_copy(cnt_v, cnt_sc.at[pl.ds(rowoff, W)])

    def _pos(r, carry):
        for j in range(8):
            v = offs_v[pl.ds(r * 128 + j * 16, 16)]
            p = jnp.clip(v - wbase, 0, W) + rowoff
            pos2d[r, pl.ds(j * 16, 16)] = p
        return carry
    lax.fori_loop(0, B // 128, _pos, 0)

    for j in range(8):
        ones_v[pl.ds(j * 16, 16)] = jnp.full((16,), 1, jnp.int32)

    plsc.subcore_barrier()

    def _hist(j, carry):
        pltpu.sync_copy(ones_v, cnt_sc.at[pos2d.at[j]], add=True)
        return carry
    lax.fori_loop(0, B // SUB, _hist, 0)
    pltpu.sync_copy(cnt_sc.at[pl.ds(rowoff, W)], cnt_v)

    def _cs(r, carry):
        for j in range(8):
            v = cnt_v[pl.ds(r * 128 + j * 16, 16)]
            seg2d[r, pl.ds(j * 16, 16)] = plsc.cumsum(v) + carry - 1
            carry = carry + jnp.sum(v)
        return carry
    lax.fori_loop(0, SEGROWS, _cs, jnp.int32(0))

    bbase = wid * BPW
    for j in range(BPW // 16):
        a = offs_v[pl.ds(bbase + j * 16, 16)]
        b2 = offs_v[pl.ds(bbase + j * 16 + 1, 16)]
        cntf = (b2 - a).astype(jnp.float32)
        inv_v[pl.ds(j * 16, 16)] = 1.0 / jnp.maximum(cntf, 1.0)
    pltpu.sync_copy(inv_v, inv_hbm.at[pl.ds(bbase, BPW)])

    def _chunk(k, carry):
        tb = wbase + k * CH
        pltpu.sync_copy(text_hbm.at[pl.ds(tb, CH)], idx_v)
        descs = [
            pltpu.async_copy(emb_hbm.at[idx_v.at[pl.ds(j * SUB, SUB)]],
                             rows_v.at[pl.ds(j * SUB, SUB)], sem)
            for j in range(NSUB)
        ]
        for d in descs:
            d.wait()
        for j in range(NSUB):
            pltpu.sync_copy(rows_v.at[pl.ds(j * SUB, SUB)],
                            acc_sc.at[seg2d.at[k * NSUB + j]], add=True)
        return carry
    lax.fori_loop(0, NCHUNK, _chunk, 0)

    plsc.subcore_barrier()

    pltpu.sync_copy(acc_sc.at[pl.ds(sid * (B // NS), B // NS)],
                    rows_v.at[pl.ds(0, B // NS)])
    rowbase = cid * B + sid * (B // NS)
    pltpu.sync_copy(rows_v.at[pl.ds(0, B // NS)],
                    part_hbm.at[pl.ds(rowbase, B // NS)])


_sc_embed = pl.kernel(
    _sc_body,
    out_type=(
        jax.ShapeDtypeStruct((NC * B, D), jnp.float32),
        jax.ShapeDtypeStruct((B,), jnp.float32),
    ),
    mesh=plsc.VectorSubcoreMesh(core_axis_name="c", subcore_axis_name="s",
                                num_cores=NC, num_subcores=NS),
    scratch_types=[
        pltpu.VMEM((B + 16,), jnp.int32),
        pltpu.VMEM((B // SUB, SUB), jnp.int32),
        pltpu.VMEM((SUB,), jnp.int32),
        pltpu.VMEM((W,), jnp.int32),
        pltpu.VMEM((SEGROWS, SUB), jnp.int32),
        pltpu.VMEM((CH,), jnp.int32),
        pltpu.VMEM((CH, D), jnp.float32),
        pltpu.VMEM((BPW,), jnp.float32),
        pltpu.VMEM_SHARED((NS * CNTW,), jnp.int32),
        pltpu.VMEM_SHARED((B, D), jnp.float32),
        pltpu.SemaphoreType.DMA,
    ],
    compiler_params=pltpu.CompilerParams(needs_layout_passes=False,
                                         use_tc_tiling_on_sc=False),
)


def _tc_body(p_ref, inv_ref, fcw_ref, bias_ref, out_ref):
    sums = p_ref[pl.ds(0, B), :] + p_ref[pl.ds(B, B), :]
    means = sums * inv_ref[...]
    out_ref[...] = lax.dot_general(
        means, fcw_ref[...], (((1,), (1,)), ((), ())),
        preferred_element_type=jnp.float32) + bias_ref[...]


_tc_head = pl.pallas_call(
    _tc_body,
    out_shape=jax.ShapeDtypeStruct((B, C), jnp.float32),
)


def kernel(text, offsets, emb_weight, fc_weight, fc_bias):
    part, inv = _sc_embed(text, offsets, emb_weight)
    return _tc_head(part, inv.reshape(B, 1), fc_weight, fc_bias.reshape(1, C))


# speedup vs baseline: 38.5267x; 1.0225x over previous
import jax
import jax.numpy as jnp
from jax import lax
from jax.experimental import pallas as pl
from jax.experimental.pallas import tpu as pltpu
from jax.experimental.pallas import tpu_sc as plsc

T = 204800
B = 4096
D = 32
C = 4
NC = 2
NS = 16
NW = NC * NS
W = T // NW
CH = 1280
NCHUNK = W // CH
SUB = 128
NSUB = CH // SUB
SEGROWS = W // SUB
CNTW = W + 8
BPW = B // NW
BPS = B // NS


def _sc_body(text_hbm, offs_hbm, emb_hbm, part_hbm, inv_hbm,
             offs_v, pos2d, ones_v, cnt_v, seg2d, idx_all, rows_a, rows_b,
             inv_v, cnt_sc, acc_sc, lsem, hsem, gsem0, gsem1, ssem0, ssem1):
    cid = lax.axis_index("c")
    sid = lax.axis_index("s")
    wid = cid * NS + sid
    wbase = wid * W

    ld_off = pltpu.async_copy(offs_hbm, offs_v.at[pl.ds(0, B)], lsem)
    ld_idx = pltpu.async_copy(text_hbm.at[pl.ds(wbase, W)], idx_all, lsem)

    zeros16f = jnp.zeros((16,), jnp.float32)
    zeros16i = jnp.zeros((16,), jnp.int32)

    def _z1(i, carry):
        rows_a[i, pl.ds(0, 16)] = zeros16f
        rows_a[i, pl.ds(16, 16)] = zeros16f
        return carry
    lax.fori_loop(0, BPS, _z1, 0)
    pltpu.sync_copy(rows_a.at[pl.ds(0, BPS)],
                    acc_sc.at[pl.ds(sid * BPS, BPS)])

    def _z2(i, carry):
        cnt_v[pl.ds(i * 16, 16)] = zeros16i
        return carry
    lax.fori_loop(0, W // 16, _z2, 0)
    rowoff = sid * CNTW
    pltpu.sync_copy(cnt_v, cnt_sc.at[pl.ds(rowoff, W)])

    def _fire_gather(k, rows, sem):
        return [
            pltpu.async_copy(emb_hbm.at[idx_all.at[pl.ds(k * CH + j * SUB, SUB)]],
                             rows.at[pl.ds(j * SUB, SUB)], sem)
            for j in range(NSUB)
        ]

    ld_idx.wait()
    gd0 = _fire_gather(0, rows_a, gsem0)

    ld_off.wait()
    offs_v[pl.ds(B, 16)] = jnp.full((16,), T, jnp.int32)

    def _pos(r, carry):
        for j in range(8):
            v = offs_v[pl.ds(r * 128 + j * 16, 16)]
            p = jnp.clip(v - wbase, 0, W) + rowoff
            pos2d[r, pl.ds(j * 16, 16)] = p
        return carry
    lax.fori_loop(0, B // 128, _pos, 0)

    for j in range(8):
        ones_v[pl.ds(j * 16, 16)] = jnp.full((16,), 1, jnp.int32)

    hd = [pltpu.async_copy(ones_v, cnt_sc.at[pos2d.at[j]], hsem, add=True)
          for j in range(B // SUB)]
    for d in hd:
        d.wait()
    pltpu.sync_copy(cnt_sc.at[pl.ds(rowoff, W)], cnt_v)

    def _cs(r, carry):
        for j in range(8):
            v = cnt_v[pl.ds(r * 128 + j * 16, 16)]
            seg2d[r, pl.ds(j * 16, 16)] = plsc.cumsum(v) + carry - 1
            carry = carry + jnp.sum(v)
        return carry
    lax.fori_loop(0, SEGROWS, _cs, jnp.int32(0))

    bbase = wid * BPW
    for j in range(BPW // 16):
        a = offs_v[pl.ds(bbase + j * 16, 16)]
        b2 = offs_v[pl.ds(bbase + j * 16 + 1, 16)]
        cntf = (b2 - a).astype(jnp.float32)
        inv_v[pl.ds(j * 16, 16)] = 1.0 / jnp.maximum(cntf, 1.0)
    pltpu.sync_copy(inv_v, inv_hbm.at[pl.ds(bbase, BPW)])

    plsc.subcore_barrier()

    rows = [rows_a, rows_b]
    gsems = [gsem0, gsem1]
    ssems = [ssem0, ssem1]
    gd = [gd0, None]
    sd = [None, None]
    for k in range(NCHUNK):
        cur = k % 2
        nxt = (k + 1) % 2
        if k + 1 < NCHUNK:
            if sd[nxt] is not None:
                for d in sd[nxt]:
                    d.wait()
            gd[nxt] = _fire_gather(k + 1, rows[nxt], gsems[nxt])
        for d in gd[cur]:
            d.wait()
        sd[cur] = [
            pltpu.async_copy(rows[cur].at[pl.ds(j * SUB, SUB)],
                             acc_sc.at[seg2d.at[k * NSUB + j]],
                             ssems[cur], add=True)
            for j in range(NSUB)
        ]
    for lst in sd:
        if lst is not None:
            for d in lst:
                d.wait()

    plsc.subcore_barrier()

    rowbase = cid * B + sid * BPS
    pltpu.sync_copy(acc_sc.at[pl.ds(sid * BPS, BPS)],
                    part_hbm.at[pl.ds(rowbase, BPS)])


_sc_embed = pl.kernel(
    _sc_body,
    out_type=(
        jax.ShapeDtypeStruct((NC * B, D), jnp.float32),
        jax.ShapeDtypeStruct((B,), jnp.float32),
    ),
    mesh=plsc.VectorSubcoreMesh(core_axis_name="c", subcore_axis_name="s",
                                num_cores=NC, num_subcores=NS),
    scratch_types=[
        pltpu.VMEM((B + 16,), jnp.int32),
        pltpu.VMEM((B // SUB, SUB), jnp.int32),
        pltpu.VMEM((SUB,), jnp.int32),
        pltpu.VMEM((W,), jnp.int32),
        pltpu.VMEM((SEGROWS, SUB), jnp.int32),
        pltpu.VMEM((W,), jnp.int32),
        pltpu.VMEM((CH, D), jnp.float32),
        pltpu.VMEM((CH, D), jnp.float32),
        pltpu.VMEM((BPW,), jnp.float32),
        pltpu.VMEM_SHARED((NS * CNTW,), jnp.int32),
        pltpu.VMEM_SHARED((B, D), jnp.float32),
        pltpu.SemaphoreType.DMA,
        pltpu.SemaphoreType.DMA,
        pltpu.SemaphoreType.DMA,
        pltpu.SemaphoreType.DMA,
        pltpu.SemaphoreType.DMA,
        pltpu.SemaphoreType.DMA,
    ],
    compiler_params=pltpu.CompilerParams(needs_layout_passes=False,
                                         use_tc_tiling_on_sc=False),
)


def _tc_body(p_ref, inv_ref, fcw_ref, bias_ref, out_ref):
    sums = p_ref[pl.ds(0, B), :] + p_ref[pl.ds(B, B), :]
    means = sums * inv_ref[...]
    out_ref[...] = lax.dot_general(
        means, fcw_ref[...], (((1,), (1,)), ((), ())),
        preferred_element_type=jnp.float32) + bias_ref[...]


_tc_head = pl.pallas_call(
    _tc_body,
    out_shape=jax.ShapeDtypeStruct((B, C), jnp.float32),
)


def kernel(text, offsets, emb_weight, fc_weight, fc_bias):
    part, inv = _sc_embed(text, offsets, emb_weight)
    return _tc_head(part, inv.reshape(B, 1), fc_weight, fc_bias.reshape(1, C))
